# baseline (speedup 1.0000x reference)
import functools

import jax
import jax.numpy as jnp
from jax import lax
from jax.experimental import pallas as pl
from jax.experimental.pallas import tpu as pltpu
from jax.experimental.pallas import tpu_sc as plsc


def kernel(_input, weights):
    bsz, seq_len = _input.shape
    embed_dim = weights.shape[1]

    info = plsc.get_sparse_core_info()
    nc, ns = info.num_cores, info.num_subcores
    nw = nc * ns
    rows_per_w = seq_len // nw
    chunk = 32
    n_chunks = rows_per_w // chunk

    mesh = plsc.VectorSubcoreMesh(core_axis_name="c", subcore_axis_name="s")

    @functools.partial(
        pl.kernel,
        mesh=mesh,
        out_type=jax.ShapeDtypeStruct((bsz * seq_len, embed_dim), jnp.float32),
        scratch_types=[
            pltpu.VMEM((chunk, embed_dim), jnp.float32),
            pltpu.VMEM((chunk, embed_dim), jnp.float32),
            pltpu.VMEM_SHARED((ns, 2, chunk, embed_dim), jnp.float32),
            pltpu.SemaphoreType.DMA,
            pltpu.SemaphoreType.DMA,
            pltpu.SemaphoreType.DMA,
            pltpu.SemaphoreType.DMA,
        ],
    )
    def k(w_hbm, out_hbm, tb0, tb1, shared, sem_ti, sem_si, sem_to, sem_so):
        cid = lax.axis_index("c")
        sid = lax.axis_index("s")
        wid = sid * nc + cid
        base = wid * rows_per_w
        tbufs = (tb0, tb1)
        ti, si, to, so = {}, {}, {}, {}

        def fire_in(i):
            start = base + i * chunk
            ti[i] = pltpu.async_copy(
                w_hbm.at[pl.ds(start, chunk)], tbufs[i % 2], sem_ti
            )
            si[i] = pltpu.async_copy(
                w_hbm.at[pl.ds(start, chunk)], shared.at[sid, i % 2], sem_si
            )

        fire_in(0)
        for i in range(n_chunks):
            if i + 1 < n_chunks:
                if i >= 1:
                    for cp in to[i - 1]:
                        cp.wait()
                    so[i - 1].wait()
                fire_in(i + 1)
            start = base + i * chunk
            ti[i].wait()
            to[i] = [
                pltpu.async_copy(
                    tbufs[i % 2],
                    out_hbm.at[pl.ds(b * seq_len + start, chunk)],
                    sem_to,
                )
                for b in range(bsz - 1)
            ]
            si[i].wait()
            so[i] = pltpu.async_copy(
                shared.at[sid, i % 2],
                out_hbm.at[pl.ds((bsz - 1) * seq_len + start, chunk)],
                sem_so,
            )
        for i in range(max(0, n_chunks - 2), n_chunks):
            for cp in to[i]:
                cp.wait()
            so[i].wait()

    out = k(weights)
    return out.reshape(bsz, seq_len, embed_dim)


# speedup vs baseline: 1.1392x; 1.1392x over previous
import functools

import jax
import jax.numpy as jnp
from jax import lax
from jax.experimental import pallas as pl
from jax.experimental.pallas import tpu as pltpu
from jax.experimental.pallas import tpu_sc as plsc


def kernel(_input, weights):
    bsz, seq_len = _input.shape
    embed_dim = weights.shape[1]

    info = plsc.get_sparse_core_info()
    nc, ns = info.num_cores, info.num_subcores
    nw = nc * ns
    rows_per_w = seq_len // nw
    chunk = 32
    n_chunks = rows_per_w // chunk

    mesh = plsc.VectorSubcoreMesh(core_axis_name="c", subcore_axis_name="s")

    @functools.partial(
        pl.kernel,
        mesh=mesh,
        out_type=jax.ShapeDtypeStruct((bsz * seq_len, embed_dim), jnp.float32),
        scratch_types=[
            pltpu.VMEM((chunk, embed_dim), jnp.float32),
            pltpu.VMEM((chunk, embed_dim), jnp.float32),
            pltpu.VMEM((chunk, embed_dim), jnp.float32),
            pltpu.SemaphoreType.DMA,
            pltpu.SemaphoreType.DMA,
        ],
    )
    def k(w_hbm, out_hbm, buf0, buf1, buf2, sem_in, sem_out):
        wid = lax.axis_index("s") * nc + lax.axis_index("c")
        base = wid * rows_per_w
        bufs = (buf0, buf1, buf2)
        nbuf = len(bufs)
        in_cp = {}
        out_cp = {}
        in_cp[0] = pltpu.async_copy(w_hbm.at[pl.ds(base, chunk)], bufs[0], sem_in)
        for i in range(n_chunks):
            if i + 1 < n_chunks:
                if i + 1 >= nbuf:
                    for cp in out_cp[i + 1 - nbuf]:
                        cp.wait()
                in_cp[i + 1] = pltpu.async_copy(
                    w_hbm.at[pl.ds(base + (i + 1) * chunk, chunk)],
                    bufs[(i + 1) % nbuf],
                    sem_in,
                )
            in_cp[i].wait()
            start = base + i * chunk
            out_cp[i] = [
                pltpu.async_copy(
                    bufs[i % nbuf],
                    out_hbm.at[pl.ds(b * seq_len + start, chunk)],
                    sem_out,
                )
                for b in range(bsz)
            ]
        for i in range(max(0, n_chunks - nbuf), n_chunks):
            for cp in out_cp[i]:
                cp.wait()

    out = k(weights)
    return out.reshape(bsz, seq_len, embed_dim)
